# baseline (speedup 1.0000x reference)
import functools

import numpy as np

import jax
import jax.numpy as jnp
from jax import lax
from jax.experimental import pallas as pl
from jax.experimental.pallas import tpu as pltpu
from jax.sharding import Mesh, PartitionSpec as P

try:
    from jax.experimental.shard_map import shard_map as _shard_map
except ImportError:
    _shard_map = jax.shard_map

_N = 4096
_M = 4096
_K = 2048
_BM = 512
_BN = 1024
_SUB = 512
_NSUB = _BN // _SUB
_JRR = _N // _BN
_JTOT = (_N + _M) // _BN
_NNK = 5

_DOT_DN = (((1,), (1,)), ((), ()))


def _hit_body(a_ref, b_ref, hit_ref, sbuf, thr_buf):
    i = pl.program_id(0)
    j = pl.program_id(1)

    a = a_ref[...]
    a_bf = a.astype(jnp.bfloat16)

    for c in range(_NSUB):
        b = b_ref[c * _SUB:(c + 1) * _SUB, :]
        b2h = 0.5 * jnp.sum(b * b, axis=1)[None, :]
        g = lax.dot_general(a_bf, b.astype(jnp.bfloat16), _DOT_DN,
                            preferred_element_type=jnp.float32)
        s = g - b2h

        @pl.when(j < _JRR)
        def _rr_phase(s=s, c=c):
            sbuf[:, pl.ds(j * _BN + c * _SUB, _SUB)] = s

        @pl.when(j >= _JRR)
        def _rg_phase(s=s, c=c):
            thr = thr_buf[...]
            colany = jnp.max((s >= thr).astype(jnp.float32), axis=0,
                             keepdims=True)
            off = (j - _JRR) * _BN + c * _SUB

            @pl.when(i == 0)
            def _init():
                hit_ref[:, pl.ds(off, _SUB)] = colany

            @pl.when(i > 0)
            def _accum():
                prev = hit_ref[:, pl.ds(off, _SUB)]
                hit_ref[:, pl.ds(off, _SUB)] = jnp.maximum(prev, colany)

    @pl.when(j == _JRR - 1)
    def _extract_threshold():
        a2h = 0.5 * jnp.sum(a * a, axis=1, keepdims=True)
        cur = sbuf[...]
        for _ in range(_NNK):
            m = jnp.max(cur, axis=1, keepdims=True)
            cur = jnp.where(cur >= m, -jnp.inf, cur)
        sel = jnp.max(cur, axis=1, keepdims=True)
        thr_buf[...] = jnp.minimum(sel, a2h)


def _hits_local(real_local, b_cat):
    rows = real_local.shape[0]
    return pl.pallas_call(
        _hit_body,
        grid=(rows // _BM, _JTOT),
        in_specs=[
            pl.BlockSpec((_BM, _K), lambda i, j: (i, 0)),
            pl.BlockSpec((_BN, _K), lambda i, j: (j, 0)),
        ],
        out_specs=pl.BlockSpec((1, _M), lambda i, j: (0, 0)),
        out_shape=jax.ShapeDtypeStruct((1, _M), jnp.float32),
        scratch_shapes=[
            pltpu.VMEM((_BM, _N), jnp.float32),
            pltpu.VMEM((_BM, 1), jnp.float32),
        ],
        compiler_params=pltpu.CompilerParams(
            dimension_semantics=("arbitrary", "arbitrary"),
        ),
        interpret=False,
    )(real_local, b_cat)


def _mean_body(h_ref, o_ref):
    o_ref[0, 0] = jnp.sum(h_ref[...]) * (1.0 / _M)


def _mean_hits(hits):
    return pl.pallas_call(
        _mean_body,
        out_specs=pl.BlockSpec(memory_space=pltpu.SMEM),
        out_shape=jax.ShapeDtypeStruct((1, 1), jnp.float32),
        interpret=False,
    )(hits)[0, 0]


@functools.partial(jax.jit)
def kernel(real_stats, gen_stats):
    b_cat = jnp.concatenate([real_stats, gen_stats], axis=0)
    devs = jax.devices()
    ncores = 1
    for c in (2, 4, 8):
        if len(devs) >= c:
            ncores = c
    mesh = Mesh(np.array(devs[:ncores]), ("x",))

    @functools.partial(
        _shard_map, mesh=mesh,
        in_specs=(P("x", None), P(None, None)),
        out_specs=P(),
        check_rep=False,
    )
    def _run(real_local, b_cat_rep):
        hit = _hits_local(real_local, b_cat_rep)
        hit = lax.pmax(hit, "x")
        return _mean_hits(hit)

    return _run(real_stats, b_cat)


# speedup vs baseline: 2.5207x; 2.5207x over previous
import functools

import jax
import jax.numpy as jnp
from jax import lax
from jax.experimental import pallas as pl
from jax.experimental.pallas import tpu as pltpu

_N = 4096
_M = 4096
_K = 2048
_BM = 512
_BN = 1024
_SUB = 512
_NSUB = _BN // _SUB
_JRR = _N // _BN
_JTOT = (_N + _M) // _BN
_NNK = 5

_DOT_DN = (((1,), (1,)), ((), ()))


def _body(a_ref, b_ref, out_ref, sbuf, thr_buf, hit_buf):
    i = pl.program_id(0)
    j = pl.program_id(1)

    a = a_ref[...]
    a_bf = a.astype(jnp.bfloat16)

    @pl.when(j == _JRR)
    def _extract_threshold():
        a2h = 0.5 * jnp.sum(a * a, axis=1, keepdims=True)
        cur = sbuf[...]
        for _ in range(_NNK):
            m = jnp.max(cur, axis=1, keepdims=True)
            cur = jnp.where(cur >= m, -jnp.inf, cur)
        sel = jnp.max(cur, axis=1, keepdims=True)
        thr_buf[...] = jnp.minimum(sel, a2h)

    for c in range(_NSUB):
        b = b_ref[c * _SUB:(c + 1) * _SUB, :]
        b2h = 0.5 * jnp.sum(b * b, axis=1)[None, :]
        g = lax.dot_general(a_bf, b.astype(jnp.bfloat16), _DOT_DN,
                            preferred_element_type=jnp.float32)
        s = g - b2h

        @pl.when(j < _JRR)
        def _rr_phase(s=s, c=c):
            sbuf[:, pl.ds(j * _BN + c * _SUB, _SUB)] = s

        @pl.when(j >= _JRR)
        def _rg_phase(s=s, c=c):
            thr = thr_buf[...]
            colany = jnp.max((s >= thr).astype(jnp.float32), axis=0,
                             keepdims=True)
            off = (j - _JRR) * _BN + c * _SUB

            @pl.when(i == 0)
            def _init():
                hit_buf[:, pl.ds(off, _SUB)] = colany

            @pl.when(i > 0)
            def _accum():
                prev = hit_buf[:, pl.ds(off, _SUB)]
                hit_buf[:, pl.ds(off, _SUB)] = jnp.maximum(prev, colany)

    @pl.when((i == _N // _BM - 1) & (j == _JTOT - 1))
    def _finish():
        out_ref[0, 0] = jnp.sum(hit_buf[...]) * (1.0 / _M)


@functools.partial(jax.jit)
def kernel(real_stats, gen_stats):
    b_cat = jnp.concatenate([real_stats, gen_stats], axis=0)
    grid = (_N // _BM, _JTOT)
    out = pl.pallas_call(
        _body,
        grid=grid,
        in_specs=[
            pl.BlockSpec((_BM, _K), lambda i, j: (i, 0)),
            pl.BlockSpec((_BN, _K), lambda i, j: (j, 0)),
        ],
        out_specs=pl.BlockSpec(memory_space=pltpu.SMEM),
        out_shape=jax.ShapeDtypeStruct((1, 1), jnp.float32),
        scratch_shapes=[
            pltpu.VMEM((_BM, _N), jnp.float32),
            pltpu.VMEM((_BM, 1), jnp.float32),
            pltpu.VMEM((1, _M), jnp.float32),
        ],
        compiler_params=pltpu.CompilerParams(
            dimension_semantics=("arbitrary", "arbitrary"),
        ),
        interpret=False,
    )(real_stats, b_cat)
    return out[0, 0]
